# Initial kernel scaffold; baseline (speedup 1.0000x reference)
import jax
import jax.numpy as jnp
from jax.experimental import pallas as pl


def kernel(atoms, edges, W, b):
    raise NotImplementedError("write your pallas kernel here")


# speedup vs baseline: 2.1914x; 2.1914x over previous
import functools

import jax
import jax.numpy as jnp
from jax import lax
from jax.experimental import pallas as pl
from jax.experimental.pallas import tpu as pltpu
from jax.experimental.pallas import tpu_sc as plsc

NC = 2
NS = 16
NW = NC * NS
GN = 16
RB = 4
SLOTS = 7


def _sc_gather_sum(atoms2, idx_grp, coeff16, n_pad):
  d = atoms2.shape[-1]
  npw = n_pad // NW
  ng = npw // GN
  assert ng % RB == 0 and ng % 8 == 0

  mesh = plsc.VectorSubcoreMesh(
      core_axis_name="c", subcore_axis_name="s", num_cores=NC, num_subcores=NS)

  @functools.partial(
      pl.kernel,
      out_type=jax.ShapeDtypeStruct((n_pad, d), jnp.float32),
      mesh=mesh,
      scratch_types=[
          pltpu.VMEM((ng, GN * SLOTS), jnp.int32),
          pltpu.VMEM((RB, GN, 16), jnp.float32),
          pltpu.VMEM((RB, GN * SLOTS, d), jnp.float32),
          pltpu.VMEM((RB, GN, d), jnp.float32),
          pltpu.SemaphoreType.DMA((RB,)),
          pltpu.SemaphoreType.DMA((RB,)),
          pltpu.SemaphoreType.DMA((RB,)),
      ],
  )
  def sc_kernel(atoms_hbm, idx_hbm, coeff_hbm, out_hbm,
                idx_v, cf_v, rows_v, out_v, gsem, csem, osem):
    wid = lax.axis_index("s") * NC + lax.axis_index("c")
    gbase = wid * ng
    nbase = wid * npw

    pltpu.sync_copy(idx_hbm.at[pl.ds(gbase, ng)], idx_v)

    def start_gather(g, slot):
      pltpu.async_copy(atoms_hbm.at[idx_v.at[g]], rows_v.at[slot],
                       gsem.at[slot])
      pltpu.async_copy(coeff_hbm.at[pl.ds(nbase + g * GN, GN)],
                       cf_v.at[slot], csem.at[slot])

    def start_store(g, slot):
      pltpu.async_copy(out_v.at[slot],
                       out_hbm.at[pl.ds(nbase + g * GN, GN)], osem.at[slot])

    def wait_gather(slot):
      pltpu.make_async_copy(atoms_hbm.at[idx_v.at[0]], rows_v.at[slot],
                            gsem.at[slot]).wait()
      pltpu.make_async_copy(coeff_hbm.at[pl.ds(nbase, GN)], cf_v.at[slot],
                            csem.at[slot]).wait()

    def wait_store(slot):
      pltpu.make_async_copy(out_v.at[slot], out_hbm.at[pl.ds(nbase, GN)],
                            osem.at[slot]).wait()

    for r in range(RB):
      start_gather(r, r)

    def compute_group(g, slot):
      def node_body(i, carry):
        row = lambda k, c: rows_v[slot, i * SLOTS + k, pl.ds(c * 16, 16)]
        spl = cf_v[slot, i, :]
        for c in range(d // 16):
          acc = row(0, c) + row(1, c)
          acc = acc + (row(2, c) + row(3, c))
          acc = acc + (row(4, c) + row(5, c))
          s_r = row(6, c)
          acc = acc + (s_r + spl * s_r)
          out_v[slot, i, pl.ds(c * 16, 16)] = acc
        return carry
      lax.fori_loop(0, GN, node_body, 0, unroll=2)

    def outer(o, carry):
      gg = o * RB
      for r in range(RB):
        g = gg + r
        wait_gather(r)

        @pl.when(o > 0)
        def _():
          wait_store(r)

        compute_group(g, r)
        start_store(g, r)

        @pl.when(g + RB < ng)
        def _():
          start_gather(g + RB, r)

      return carry

    lax.fori_loop(0, ng // RB, outer, 0)
    for r in range(RB):
      wait_store(r)

  return sc_kernel(atoms2, idx_grp, coeff16)


def _tc_degree_linear(summed, deg_f, w, b8, n_pad, tile):
  d = summed.shape[-1]
  ndeg = w.shape[0]

  def body(x_ref, dg_ref, w_ref, b_ref, o_ref):
    x = x_ref[...]
    dg = dg_ref[...]
    acc = jnp.zeros((tile, d), jnp.float32)
    for k in range(ndeg):
      y = lax.dot_general(x, w_ref[k], (((1,), (0,)), ((), ())),
                          preferred_element_type=jnp.float32)
      y = jnp.maximum(y + b_ref[k, :][None, :], 0.0)
      acc = acc + jnp.where(dg == float(k), y, 0.0)
    o_ref[...] = acc

  return pl.pallas_call(
      body,
      grid=(n_pad // tile,),
      in_specs=[
          pl.BlockSpec((tile, d), lambda i: (i, 0)),
          pl.BlockSpec((tile, 1), lambda i: (i, 0)),
          pl.BlockSpec((ndeg, d, d), lambda i: (0, 0, 0)),
          pl.BlockSpec((8, d), lambda i: (0, 0)),
      ],
      out_specs=pl.BlockSpec((tile, d), lambda i: (i, 0)),
      out_shape=jax.ShapeDtypeStruct((n_pad, d), jnp.float32),
  )(summed, deg_f, w, b8)


def kernel(atoms, edges, W, b):
  bsz, n, d = atoms.shape
  k = edges.shape[-1]

  align = NW * GN * 8
  n_pad = ((n + align - 1) // align) * align
  tile = 1024
  assert n_pad % tile == 0 and bsz == 1

  atoms2 = atoms[0]
  e = edges[0]
  self_idx = jnp.arange(n, dtype=jnp.int32)
  deg = (e != -1).sum(-1).astype(jnp.int32)
  idx7 = jnp.concatenate(
      [jnp.where(e == -1, self_idx[:, None], e), self_idx[:, None]], axis=1)
  idx7 = jnp.pad(idx7, ((0, n_pad - n), (0, 0)))
  idx_grp = idx7.reshape(n_pad // GN, GN * SLOTS)
  coeff = jnp.pad((deg - k).astype(jnp.float32), (0, n_pad - n))
  coeff16 = jnp.broadcast_to(coeff[:, None], (n_pad, 16))

  summed = _sc_gather_sum(atoms2, idx_grp, coeff16, n_pad)

  deg_f = jnp.pad(deg.astype(jnp.float32), (0, n_pad - n))[:, None]
  b8 = jnp.pad(b, ((0, 8 - b.shape[0]), (0, 0)))
  out = _tc_degree_linear(summed, deg_f, W, b8, n_pad, tile)
  return out[:n][None]


# speedup vs baseline: 2.3190x; 1.0583x over previous
import functools

import jax
import jax.numpy as jnp
from jax import lax
from jax.experimental import pallas as pl
from jax.experimental.pallas import tpu as pltpu
from jax.experimental.pallas import tpu_sc as plsc

NC = 2
NS = 16
NW = NC * NS
GN = 16
RB = 4
SLOTS = 7


def _sc_gather_sum(atoms2, idx_grp, coeff16, n_pad, ng0, ng1):
  d = atoms2.shape[-1]
  assert NS * (ng0 + ng1) * GN == n_pad
  for ngc in (ng0, ng1):
    assert ngc % RB == 0 and ngc % 8 == 0 and ngc >= RB
  ngmax = max(ng0, ng1)

  mesh = plsc.VectorSubcoreMesh(
      core_axis_name="c", subcore_axis_name="s", num_cores=NC, num_subcores=NS)

  @functools.partial(
      pl.kernel,
      out_type=jax.ShapeDtypeStruct((n_pad, d), jnp.float32),
      mesh=mesh,
      scratch_types=[
          pltpu.VMEM((ngmax, GN * SLOTS), jnp.int32),
          pltpu.VMEM((RB, GN, 16), jnp.float32),
          pltpu.VMEM((RB, GN * SLOTS, d), jnp.float32),
          pltpu.VMEM((RB, GN, d), jnp.float32),
          pltpu.SemaphoreType.DMA((RB,)),
          pltpu.SemaphoreType.DMA((RB,)),
          pltpu.SemaphoreType.DMA((RB,)),
      ],
  )
  def sc_kernel(atoms_hbm, idx_hbm, coeff_hbm, out_hbm,
                idx_v, cf_v, rows_v, out_v, gsem, csem, osem):
    c = lax.axis_index("c")
    s = lax.axis_index("s")
    ng = jnp.where(c == 0, ng0, ng1)
    gbase = jnp.where(c == 0, s * ng0, NS * ng0 + s * ng1)
    nbase = gbase * GN

    pltpu.sync_copy(idx_hbm.at[pl.ds(gbase, ngmax)], idx_v)

    def start_gather(g, slot):
      pltpu.async_copy(atoms_hbm.at[idx_v.at[g]], rows_v.at[slot],
                       gsem.at[slot])
      pltpu.async_copy(coeff_hbm.at[pl.ds(nbase + g * GN, GN)],
                       cf_v.at[slot], csem.at[slot])

    def start_store(g, slot):
      pltpu.async_copy(out_v.at[slot],
                       out_hbm.at[pl.ds(nbase + g * GN, GN)], osem.at[slot])

    def wait_gather(slot):
      pltpu.make_async_copy(atoms_hbm.at[idx_v.at[0]], rows_v.at[slot],
                            gsem.at[slot]).wait()
      pltpu.make_async_copy(coeff_hbm.at[pl.ds(nbase, GN)], cf_v.at[slot],
                            csem.at[slot]).wait()

    def wait_store(slot):
      pltpu.make_async_copy(out_v.at[slot], out_hbm.at[pl.ds(nbase, GN)],
                            osem.at[slot]).wait()

    for r in range(RB):
      start_gather(r, r)

    def compute_group(g, slot):
      def node_body(i, carry):
        row = lambda k, c: rows_v[slot, i * SLOTS + k, pl.ds(c * 16, 16)]
        spl = cf_v[slot, i, :]
        for c in range(d // 16):
          acc = row(0, c) + row(1, c)
          acc = acc + (row(2, c) + row(3, c))
          acc = acc + (row(4, c) + row(5, c))
          s_r = row(6, c)
          acc = acc + (s_r + spl * s_r)
          out_v[slot, i, pl.ds(c * 16, 16)] = acc
        return carry
      lax.fori_loop(0, GN, node_body, 0, unroll=2)

    def outer(o, carry):
      gg = o * RB
      for r in range(RB):
        g = gg + r
        wait_gather(r)

        @pl.when(o > 0)
        def _():
          wait_store(r)

        compute_group(g, r)
        start_store(g, r)

        @pl.when(g + RB < ng)
        def _():
          start_gather(g + RB, r)

      return carry

    lax.fori_loop(0, lax.div(ng, RB), outer, 0)
    for r in range(RB):
      wait_store(r)

  return sc_kernel(atoms2, idx_grp, coeff16)


def _tc_degree_linear(summed, deg_f, w, b8, n_pad, tile):
  d = summed.shape[-1]
  ndeg = w.shape[0]

  def body(x_ref, dg_ref, w_ref, b_ref, o_ref):
    x = x_ref[...]
    dg = dg_ref[...]
    acc = jnp.zeros((tile, d), jnp.float32)
    for k in range(ndeg):
      y = lax.dot_general(x, w_ref[k], (((1,), (0,)), ((), ())),
                          preferred_element_type=jnp.float32)
      y = jnp.maximum(y + b_ref[k, :][None, :], 0.0)
      acc = acc + jnp.where(dg == float(k), y, 0.0)
    o_ref[...] = acc

  return pl.pallas_call(
      body,
      grid=(n_pad // tile,),
      in_specs=[
          pl.BlockSpec((tile, d), lambda i: (i, 0)),
          pl.BlockSpec((tile, 1), lambda i: (i, 0)),
          pl.BlockSpec((ndeg, d, d), lambda i: (0, 0, 0)),
          pl.BlockSpec((8, d), lambda i: (0, 0)),
      ],
      out_specs=pl.BlockSpec((tile, d), lambda i: (i, 0)),
      out_shape=jax.ShapeDtypeStruct((n_pad, d), jnp.float32),
  )(summed, deg_f, w, b8)


def kernel(atoms, edges, W, b):
  bsz, n, d = atoms.shape
  k = edges.shape[-1]

  align = NW * GN * 8
  n_pad = ((n + align - 1) // align) * align
  tile = 1024
  assert n_pad % tile == 0 and bsz == 1

  atoms2 = atoms[0]
  e = edges[0]
  self_idx = jnp.arange(n, dtype=jnp.int32)
  deg = (e != -1).sum(-1).astype(jnp.int32)
  idx7 = jnp.concatenate(
      [jnp.where(e == -1, self_idx[:, None], e), self_idx[:, None]], axis=1)
  idx7 = jnp.pad(idx7, ((0, n_pad - n), (0, 0)))
  coeff = jnp.pad((deg - k).astype(jnp.float32), (0, n_pad - n))
  coeff16 = jnp.broadcast_to(coeff[:, None], (n_pad, 16))

  ng_tot = n_pad // (NS * GN)
  ng0 = (ng_tot * 4 // 5) // 8 * 8
  ng1 = ng_tot - ng0
  idx_grp = idx7.reshape(n_pad // GN, GN * SLOTS)
  idx_grp = jnp.pad(idx_grp, ((0, max(ng0, ng1)), (0, 0)))

  summed = _sc_gather_sum(atoms2, idx_grp, coeff16, n_pad, ng0, ng1)

  deg_f = jnp.pad(deg.astype(jnp.float32), (0, n_pad - n))[:, None]
  b8 = jnp.pad(b, ((0, 8 - b.shape[0]), (0, 0)))
  out = _tc_degree_linear(summed, deg_f, W, b8, n_pad, tile)
  return out[:n][None]


# speedup vs baseline: 2.3232x; 1.0018x over previous
import functools

import jax
import jax.numpy as jnp
from jax import lax
from jax.experimental import pallas as pl
from jax.experimental.pallas import tpu as pltpu
from jax.experimental.pallas import tpu_sc as plsc

NC = 2
NS = 16
NW = NC * NS
GN = 16
RB = 4
SLOTS = 7


def _sc_gather_sum(atoms2, idx_grp, coeff16, n_pad, ng0, ng1):
  d = atoms2.shape[-1]
  assert NS * (ng0 + ng1) * GN == n_pad
  for ngc in (ng0, ng1):
    assert ngc % RB == 0 and ngc % 8 == 0 and ngc >= RB
  ngmax = max(ng0, ng1)

  mesh = plsc.VectorSubcoreMesh(
      core_axis_name="c", subcore_axis_name="s", num_cores=NC, num_subcores=NS)

  @functools.partial(
      pl.kernel,
      out_type=jax.ShapeDtypeStruct((n_pad, d), jnp.float32),
      mesh=mesh,
      scratch_types=[
          pltpu.VMEM((ngmax, GN * SLOTS), jnp.int32),
          pltpu.VMEM((RB, GN, 16), jnp.float32),
          pltpu.VMEM((RB, GN * SLOTS, d), jnp.float32),
          pltpu.VMEM((RB, GN, d), jnp.float32),
          pltpu.SemaphoreType.DMA((RB,)),
          pltpu.SemaphoreType.DMA((RB,)),
          pltpu.SemaphoreType.DMA((RB,)),
      ],
  )
  def sc_kernel(atoms_hbm, idx_hbm, coeff_hbm, out_hbm,
                idx_v, cf_v, rows_v, out_v, gsem, csem, osem):
    c = lax.axis_index("c")
    s = lax.axis_index("s")
    ng = jnp.where(c == 0, ng0, ng1)
    gbase = jnp.where(c == 0, s * ng0, NS * ng0 + s * ng1)
    nbase = gbase * GN

    with jax.named_scope("stage_idx"):
      pltpu.sync_copy(idx_hbm.at[pl.ds(gbase, ngmax)], idx_v)

    def start_gather(g, slot):
      pltpu.async_copy(atoms_hbm.at[idx_v.at[g]], rows_v.at[slot],
                       gsem.at[slot])
      pltpu.async_copy(coeff_hbm.at[pl.ds(nbase + g * GN, GN)],
                       cf_v.at[slot], csem.at[slot])

    def start_store(g, slot):
      pltpu.async_copy(out_v.at[slot],
                       out_hbm.at[pl.ds(nbase + g * GN, GN)], osem.at[slot])

    def wait_gather(slot):
      pltpu.make_async_copy(atoms_hbm.at[idx_v.at[0]], rows_v.at[slot],
                            gsem.at[slot]).wait()
      pltpu.make_async_copy(coeff_hbm.at[pl.ds(nbase, GN)], cf_v.at[slot],
                            csem.at[slot]).wait()

    def wait_store(slot):
      pltpu.make_async_copy(out_v.at[slot], out_hbm.at[pl.ds(nbase, GN)],
                            osem.at[slot]).wait()

    with jax.named_scope("prime_ring"):
      for r in range(RB):
        start_gather(r, r)

    def compute_group(g, slot):
      def node_body(i, carry):
        row = lambda k, c: rows_v[slot, i * SLOTS + k, pl.ds(c * 16, 16)]
        spl = cf_v[slot, i, :]
        for c in range(d // 16):
          acc = row(0, c) + row(1, c)
          acc = acc + (row(2, c) + row(3, c))
          acc = acc + (row(4, c) + row(5, c))
          s_r = row(6, c)
          acc = acc + (s_r + spl * s_r)
          out_v[slot, i, pl.ds(c * 16, 16)] = acc
        return carry
      lax.fori_loop(0, GN, node_body, 0, unroll=2)

    def outer(o, carry):
      gg = o * RB
      for r in range(RB):
        g = gg + r
        wait_gather(r)

        @pl.when(o > 0)
        def _():
          wait_store(r)

        compute_group(g, r)
        start_store(g, r)

        @pl.when(g + RB < ng)
        def _():
          start_gather(g + RB, r)

      return carry

    with jax.named_scope("mainloop"):
      lax.fori_loop(0, lax.div(ng, RB), outer, 0)
    with jax.named_scope("drain"):
      for r in range(RB):
        wait_store(r)

  return sc_kernel(atoms2, idx_grp, coeff16)


def _tc_degree_linear(summed, deg_f, w, b8, n_pad, tile):
  d = summed.shape[-1]
  ndeg = w.shape[0]

  def body(x_ref, dg_ref, w_ref, b_ref, o_ref):
    x = x_ref[...]
    dg = dg_ref[...]
    acc = jnp.zeros((tile, d), jnp.float32)
    for k in range(ndeg):
      y = lax.dot_general(x, w_ref[k], (((1,), (0,)), ((), ())),
                          preferred_element_type=jnp.float32)
      y = jnp.maximum(y + b_ref[k, :][None, :], 0.0)
      acc = acc + jnp.where(dg == float(k), y, 0.0)
    o_ref[...] = acc

  return pl.pallas_call(
      body,
      grid=(n_pad // tile,),
      in_specs=[
          pl.BlockSpec((tile, d), lambda i: (i, 0)),
          pl.BlockSpec((tile, 1), lambda i: (i, 0)),
          pl.BlockSpec((ndeg, d, d), lambda i: (0, 0, 0)),
          pl.BlockSpec((8, d), lambda i: (0, 0)),
      ],
      out_specs=pl.BlockSpec((tile, d), lambda i: (i, 0)),
      out_shape=jax.ShapeDtypeStruct((n_pad, d), jnp.float32),
  )(summed, deg_f, w, b8)


def kernel(atoms, edges, W, b):
  bsz, n, d = atoms.shape
  k = edges.shape[-1]

  align = NW * GN * 8
  n_pad = ((n + align - 1) // align) * align
  tile = 1024
  assert n_pad % tile == 0 and bsz == 1

  atoms2 = atoms[0]
  e = edges[0]
  self_idx = jnp.arange(n, dtype=jnp.int32)
  deg = (e != -1).sum(-1).astype(jnp.int32)
  idx7 = jnp.concatenate(
      [jnp.where(e == -1, self_idx[:, None], e), self_idx[:, None]], axis=1)
  idx7 = jnp.pad(idx7, ((0, n_pad - n), (0, 0)))
  coeff = jnp.pad((deg - k).astype(jnp.float32), (0, n_pad - n))
  coeff16 = jnp.broadcast_to(coeff[:, None], (n_pad, 16))

  ng_tot = n_pad // (NS * GN)
  ng0 = (ng_tot * 4 // 5) // 8 * 8
  ng1 = ng_tot - ng0
  idx_grp = idx7.reshape(n_pad // GN, GN * SLOTS)
  idx_grp = jnp.pad(idx_grp, ((0, max(ng0, ng1)), (0, 0)))

  summed = _sc_gather_sum(atoms2, idx_grp, coeff16, n_pad, ng0, ng1)

  deg_f = jnp.pad(deg.astype(jnp.float32), (0, n_pad - n))[:, None]
  b8 = jnp.pad(b, ((0, 8 - b.shape[0]), (0, 0)))
  out = _tc_degree_linear(summed, deg_f, W, b8, n_pad, tile)
  return out[:n][None]


# speedup vs baseline: 5.8533x; 2.5195x over previous
import functools

import jax
import jax.numpy as jnp
from jax import lax
from jax.experimental import pallas as pl
from jax.experimental.pallas import tpu as pltpu
from jax.experimental.pallas import tpu_sc as plsc

NC = 2
NS = 16
NW = NC * NS
GN = 16
RB = 4
SLOTS = 7


def _sc_gather_sum(atoms2, idx_grp, coeff16, n_pad, ng0, ng1):
  d = atoms2.shape[-1]
  assert NS * (ng0 + ng1) * GN == n_pad
  for ngc in (ng0, ng1):
    assert ngc % RB == 0 and ngc % 8 == 0 and ngc >= RB
  ngmax = max(ng0, ng1)

  mesh = plsc.VectorSubcoreMesh(
      core_axis_name="c", subcore_axis_name="s", num_cores=NC, num_subcores=NS)

  @functools.partial(
      pl.kernel,
      out_type=jax.ShapeDtypeStruct((n_pad, d), jnp.float32),
      mesh=mesh,
      scratch_types=[
          pltpu.VMEM((ngmax, GN * SLOTS), jnp.int32),
          pltpu.VMEM((RB, GN, 16), jnp.float32),
          pltpu.VMEM((RB, GN * SLOTS, d), jnp.float32),
          pltpu.VMEM((RB, GN, d), jnp.float32),
          pltpu.SemaphoreType.DMA((RB,)),
          pltpu.SemaphoreType.DMA((RB,)),
          pltpu.SemaphoreType.DMA((RB,)),
      ],
  )
  def sc_kernel(atoms_hbm, idx_hbm, coeff_hbm, out_hbm,
                idx_v, cf_v, rows_v, out_v, gsem, csem, osem):
    c = lax.axis_index("c")
    s = lax.axis_index("s")
    ng = jnp.where(c == 0, ng0, ng1)
    gbase = jnp.where(c == 0, s * ng0, NS * ng0 + s * ng1)
    nbase = gbase * GN

    with jax.named_scope("stage_idx"):
      pltpu.sync_copy(idx_hbm.at[pl.ds(gbase, ngmax)], idx_v)

    def start_gather(g, slot):
      pltpu.async_copy(atoms_hbm.at[idx_v.at[g]], rows_v.at[slot],
                       gsem.at[slot])
      pltpu.async_copy(coeff_hbm.at[pl.ds(nbase + g * GN, GN)],
                       cf_v.at[slot], csem.at[slot])

    def start_store(g, slot):
      pltpu.async_copy(out_v.at[slot],
                       out_hbm.at[pl.ds(nbase + g * GN, GN)], osem.at[slot])

    def wait_gather(slot):
      pltpu.make_async_copy(atoms_hbm.at[idx_v.at[0]], rows_v.at[slot],
                            gsem.at[slot]).wait()
      pltpu.make_async_copy(coeff_hbm.at[pl.ds(nbase, GN)], cf_v.at[slot],
                            csem.at[slot]).wait()

    def wait_store(slot):
      pltpu.make_async_copy(out_v.at[slot], out_hbm.at[pl.ds(nbase, GN)],
                            osem.at[slot]).wait()

    with jax.named_scope("prime_ring"):
      for r in range(RB):
        start_gather(r, r)

    def compute_group(g, slot):
      def node_body(i, carry):
        row = lambda k, c: rows_v[slot, i * SLOTS + k, pl.ds(c * 16, 16)]
        spl = cf_v[slot, i, :]
        for c in range(d // 16):
          acc = row(0, c) + row(1, c)
          acc = acc + (row(2, c) + row(3, c))
          acc = acc + (row(4, c) + row(5, c))
          s_r = row(6, c)
          acc = acc + (s_r + spl * s_r)
          out_v[slot, i, pl.ds(c * 16, 16)] = acc
        return carry
      lax.fori_loop(0, GN, node_body, 0, unroll=2)

    def outer(o, carry):
      gg = o * RB
      for r in range(RB):
        g = gg + r
        wait_gather(r)

        @pl.when(o > 0)
        def _():
          wait_store(r)

        compute_group(g, r)
        start_store(g, r)

        @pl.when(g + RB < ng)
        def _():
          start_gather(g + RB, r)

      return carry

    with jax.named_scope("mainloop"):
      lax.fori_loop(0, lax.div(ng, RB), outer, 0)
    with jax.named_scope("drain"):
      for r in range(RB):
        wait_store(r)

  return sc_kernel(atoms2, idx_grp, coeff16)


def _tc_degree_linear(summed, deg_f, w, b8, n_pad, tile):
  d = summed.shape[-1]
  ndeg = w.shape[0]

  def body(x_ref, dg_ref, w_ref, b_ref, o_ref):
    x = x_ref[...]
    dg = dg_ref[...]
    acc = jnp.zeros((tile, d), jnp.float32)
    for k in range(ndeg):
      y = lax.dot_general(x, w_ref[k], (((1,), (0,)), ((), ())),
                          preferred_element_type=jnp.float32)
      y = jnp.maximum(y + b_ref[k, :][None, :], 0.0)
      acc = acc + jnp.where(dg == float(k), y, 0.0)
    o_ref[...] = acc

  return pl.pallas_call(
      body,
      grid=(n_pad // tile,),
      in_specs=[
          pl.BlockSpec((tile, d), lambda i: (i, 0)),
          pl.BlockSpec((tile, 1), lambda i: (i, 0)),
          pl.BlockSpec((ndeg, d, d), lambda i: (0, 0, 0)),
          pl.BlockSpec((8, d), lambda i: (0, 0)),
      ],
      out_specs=pl.BlockSpec((tile, d), lambda i: (i, 0)),
      out_shape=jax.ShapeDtypeStruct((n_pad, d), jnp.float32),
  )(summed, deg_f, w, b8)


def kernel(atoms, edges, W, b):
  bsz, n, d = atoms.shape
  k = edges.shape[-1]

  align = NW * GN * 8
  n_pad = ((n + align - 1) // align) * align
  tile = 1024
  assert n_pad % tile == 0 and bsz == 1

  atoms2 = atoms[0]
  e = edges[0]
  self_idx = jnp.arange(n, dtype=jnp.int32)
  deg = (e != -1).sum(-1).astype(jnp.int32)
  idx7 = jnp.concatenate(
      [jnp.where(e == -1, self_idx[:, None], e), self_idx[:, None]], axis=1)
  pad_rows = (jnp.arange(n, n_pad, dtype=jnp.int32) % n)[:, None]
  idx7 = jnp.concatenate([idx7, jnp.broadcast_to(pad_rows, (n_pad - n, SLOTS))])
  coeff = jnp.pad((deg - k).astype(jnp.float32), (0, n_pad - n))
  coeff16 = jnp.broadcast_to(coeff[:, None], (n_pad, 16))

  ng_tot = n_pad // (NS * GN)
  ng0 = ng_tot // 2
  ng1 = ng_tot - ng0
  idx_grp = idx7.reshape(n_pad // GN, GN * SLOTS)
  idx_grp = jnp.pad(idx_grp, ((0, max(ng0, ng1)), (0, 0)))

  summed = _sc_gather_sum(atoms2, idx_grp, coeff16, n_pad, ng0, ng1)

  deg_f = jnp.pad(deg.astype(jnp.float32), (0, n_pad - n))[:, None]
  b8 = jnp.pad(b, ((0, 8 - b.shape[0]), (0, 0)))
  out = _tc_degree_linear(summed, deg_f, W, b8, n_pad, tile)
  return out[:n][None]


# speedup vs baseline: 5.8809x; 1.0047x over previous
import functools

import jax
import jax.numpy as jnp
from jax import lax
from jax.experimental import pallas as pl
from jax.experimental.pallas import tpu as pltpu
from jax.experimental.pallas import tpu_sc as plsc

NC = 2
NS = 16
NW = NC * NS
GN = 16
RB = 4
SLOTS = 7


def _sc_gather_sum(atoms2, idx_grp, coeff16, n_pad, ng0, ng1):
  d = atoms2.shape[-1]
  assert NS * (ng0 + ng1) * GN == n_pad
  for ngc in (ng0, ng1):
    assert ngc % RB == 0 and ngc % 8 == 0 and ngc >= RB
  ngmax = max(ng0, ng1)

  mesh = plsc.VectorSubcoreMesh(
      core_axis_name="c", subcore_axis_name="s", num_cores=NC, num_subcores=NS)

  @functools.partial(
      pl.kernel,
      out_type=jax.ShapeDtypeStruct((n_pad, d), jnp.float32),
      mesh=mesh,
      scratch_types=[
          pltpu.VMEM((ngmax, GN * SLOTS), jnp.int32),
          pltpu.VMEM((RB, GN * 16), jnp.float32),
          pltpu.VMEM((RB, GN * SLOTS, d), jnp.float32),
          pltpu.VMEM((RB, GN, d), jnp.float32),
          pltpu.SemaphoreType.DMA((RB,)),
          pltpu.SemaphoreType.DMA((RB,)),
          pltpu.SemaphoreType.DMA((RB,)),
      ],
  )
  def sc_kernel(atoms_hbm, idx_hbm, coeff_hbm, out_hbm,
                idx_v, cf_v, rows_v, out_v, gsem, csem, osem):
    c = lax.axis_index("c")
    s = lax.axis_index("s")
    ng = jnp.where(c == 0, ng0, ng1)
    gbase = jnp.where(c == 0, s * ng0, NS * ng0 + s * ng1)
    nbase = gbase * GN

    with jax.named_scope("stage_idx"):
      pltpu.sync_copy(idx_hbm.at[pl.ds(gbase, ngmax)], idx_v)

    def start_gather(g, slot):
      pltpu.async_copy(atoms_hbm.at[idx_v.at[g]], rows_v.at[slot],
                       gsem.at[slot])
      pltpu.async_copy(coeff_hbm.at[pl.ds((nbase + g * GN) * 16, GN * 16)],
                       cf_v.at[slot], csem.at[slot])

    def start_store(g, slot):
      pltpu.async_copy(out_v.at[slot],
                       out_hbm.at[pl.ds(nbase + g * GN, GN)], osem.at[slot])

    def wait_gather(slot):
      pltpu.make_async_copy(atoms_hbm.at[idx_v.at[0]], rows_v.at[slot],
                            gsem.at[slot]).wait()
      pltpu.make_async_copy(coeff_hbm.at[pl.ds(nbase, GN * 16)], cf_v.at[slot],
                            csem.at[slot]).wait()

    def wait_store(slot):
      pltpu.make_async_copy(out_v.at[slot], out_hbm.at[pl.ds(nbase, GN)],
                            osem.at[slot]).wait()

    with jax.named_scope("prime_ring"):
      for r in range(RB):
        start_gather(r, r)

    def compute_group(g, slot):
      def node_body(i, carry):
        row = lambda k, c: rows_v[slot, i * SLOTS + k, pl.ds(c * 16, 16)]
        spl = cf_v[slot, pl.ds(i * 16, 16)]
        for c in range(d // 16):
          acc = row(0, c) + row(1, c)
          acc = acc + (row(2, c) + row(3, c))
          acc = acc + (row(4, c) + row(5, c))
          s_r = row(6, c)
          acc = acc + (s_r + spl * s_r)
          out_v[slot, i, pl.ds(c * 16, 16)] = acc
        return carry
      lax.fori_loop(0, GN, node_body, 0, unroll=2)

    def outer(o, carry):
      gg = o * RB
      for r in range(RB):
        g = gg + r
        wait_gather(r)

        @pl.when(o > 0)
        def _():
          wait_store(r)

        compute_group(g, r)
        start_store(g, r)

        @pl.when(g + RB < ng)
        def _():
          start_gather(g + RB, r)

      return carry

    with jax.named_scope("mainloop"):
      lax.fori_loop(0, lax.div(ng, RB), outer, 0)
    with jax.named_scope("drain"):
      for r in range(RB):
        wait_store(r)

  return sc_kernel(atoms2, idx_grp, coeff16)


def _tc_degree_linear(summed, deg_f, w, b8, n_out, tile):
  d = summed.shape[-1]
  ndeg = w.shape[0]

  def body(x_ref, dg_ref, w_ref, b_ref, o_ref):
    x = x_ref[...].astype(jnp.bfloat16)
    dg = dg_ref[...]
    acc = jnp.zeros((tile, d), jnp.float32)
    for k in range(ndeg):
      y = lax.dot_general(x, w_ref[k].astype(jnp.bfloat16),
                          (((1,), (0,)), ((), ())),
                          preferred_element_type=jnp.float32)
      y = jnp.maximum(y + b_ref[k, :][None, :], 0.0)
      acc = acc + jnp.where(dg == float(k), y, 0.0)
    o_ref[...] = acc

  grid = (n_out + tile - 1) // tile
  return pl.pallas_call(
      body,
      grid=(grid,),
      in_specs=[
          pl.BlockSpec((tile, d), lambda i: (i, 0)),
          pl.BlockSpec((tile, 1), lambda i: (i, 0)),
          pl.BlockSpec((ndeg, d, d), lambda i: (0, 0, 0)),
          pl.BlockSpec((8, d), lambda i: (0, 0)),
      ],
      out_specs=pl.BlockSpec((tile, d), lambda i: (i, 0)),
      out_shape=jax.ShapeDtypeStruct((n_out, d), jnp.float32),
  )(summed, deg_f, w, b8)


def kernel(atoms, edges, W, b):
  bsz, n, d = atoms.shape
  k = edges.shape[-1]

  align = NW * GN * 8
  n_pad = ((n + align - 1) // align) * align
  tile = 1024
  assert n_pad % tile == 0 and bsz == 1

  atoms2 = atoms[0]
  e = edges[0]
  self_idx = jnp.arange(n, dtype=jnp.int32)
  deg = (e != -1).sum(-1).astype(jnp.int32)
  idx7 = jnp.concatenate(
      [jnp.where(e == -1, self_idx[:, None], e), self_idx[:, None]], axis=1)
  pad_rows = (jnp.arange(n, n_pad, dtype=jnp.int32) % n)[:, None]
  idx7 = jnp.concatenate([idx7, jnp.broadcast_to(pad_rows, (n_pad - n, SLOTS))])
  coeff = jnp.pad((deg - k).astype(jnp.float32), (0, n_pad - n))
  coeff16 = jnp.broadcast_to(coeff[:, None], (n_pad, 16)).reshape(n_pad * 16)

  ng_tot = n_pad // (NS * GN)
  ng0 = ng_tot // 2
  ng1 = ng_tot - ng0
  idx_grp = idx7.reshape(n_pad // GN, GN * SLOTS)
  idx_grp = jnp.pad(idx_grp, ((0, max(ng0, ng1)), (0, 0)))

  summed = _sc_gather_sum(atoms2, idx_grp, coeff16, n_pad, ng0, ng1)

  deg_f = jnp.pad(deg.astype(jnp.float32), (0, n_pad - n))[:, None]
  b8 = jnp.pad(b, ((0, 8 - b.shape[0]), (0, 0)))
  out = _tc_degree_linear(summed, deg_f, W, b8, n, tile)
  return out[None]


# speedup vs baseline: 9.1601x; 1.5576x over previous
import functools

import jax
import jax.numpy as jnp
from jax import lax
from jax.experimental import pallas as pl
from jax.experimental.pallas import tpu as pltpu
from jax.experimental.pallas import tpu_sc as plsc

NC = 2
NS = 16
NW = NC * NS
GN = 16
RB = 4


def _sc_gather_sum(atoms2, edges_t, n_pad):
  d = atoms2.shape[-1]
  k = edges_t.shape[0]
  npw = n_pad // NW
  ng = npw // GN
  assert ng % RB == 0 and npw % 128 == 0
  gl = GN * k

  mesh = plsc.VectorSubcoreMesh(
      core_axis_name="c", subcore_axis_name="s", num_cores=NC, num_subcores=NS)

  @functools.partial(
      pl.kernel,
      out_type=jax.ShapeDtypeStruct((n_pad, d), jnp.float32),
      mesh=mesh,
      scratch_types=[
          pltpu.VMEM((k, npw), jnp.int32),
          pltpu.VMEM((ng * gl,), jnp.int32),
          pltpu.VMEM((RB, gl, d), jnp.float32),
          pltpu.VMEM((RB, GN, d), jnp.float32),
          pltpu.SemaphoreType.DMA((RB,)),
          pltpu.SemaphoreType.DMA((RB,)),
      ],
  )
  def sc_kernel(atoms_hbm, edges_hbm, out_hbm,
                edges_v, idx_v, rows_v, out_v, gsem, osem):
    c = lax.axis_index("c")
    s = lax.axis_index("s")
    wid = s * NC + c
    nbase = wid * npw

    with jax.named_scope("stage_edges"):
      pltpu.sync_copy(edges_hbm.at[:, pl.ds(nbase, npw)], edges_v)

    with jax.named_scope("assemble_idx"):
      def asm(g, carry):
        self16 = (jnp.full((16,), nbase + g * GN, jnp.int32)
                  + lax.iota(jnp.int32, 16))
        for kk in range(k):
          ev = edges_v[kk, pl.ds(g * GN, GN)]
          idx_v[pl.ds(g * gl + kk * GN, GN)] = jnp.where(ev < 0, self16, ev)
        return carry
      lax.fori_loop(0, ng, asm, 0, unroll=2)

    def start_gather(g, slot):
      pltpu.async_copy(atoms_hbm.at[idx_v.at[pl.ds(g * gl, gl)]],
                       rows_v.at[slot], gsem.at[slot])

    def start_store(g, slot):
      pltpu.async_copy(out_v.at[slot],
                       out_hbm.at[pl.ds(nbase + g * GN, GN)], osem.at[slot])

    def wait_gather(slot):
      pltpu.make_async_copy(atoms_hbm.at[idx_v.at[pl.ds(0, gl)]],
                            rows_v.at[slot], gsem.at[slot]).wait()

    def wait_store(slot):
      pltpu.make_async_copy(out_v.at[slot], out_hbm.at[pl.ds(nbase, GN)],
                            osem.at[slot]).wait()

    with jax.named_scope("prime_ring"):
      for r in range(RB):
        start_gather(r, r)

    def compute_group(g, slot):
      def node_body(i, carry):
        row = lambda kk, cc: rows_v[slot, kk * GN + i, pl.ds(cc * 16, 16)]
        for cc in range(d // 16):
          acc = row(0, cc) + row(1, cc)
          acc2 = row(2, cc) + row(3, cc)
          acc3 = row(4, cc) + row(5, cc)
          out_v[slot, i, pl.ds(cc * 16, 16)] = acc + (acc2 + acc3)
        return carry
      lax.fori_loop(0, GN, node_body, 0, unroll=2)

    def outer(o, carry):
      gg = o * RB
      for r in range(RB):
        g = gg + r
        wait_gather(r)

        @pl.when(o > 0)
        def _():
          wait_store(r)

        compute_group(g, r)
        start_store(g, r)

        @pl.when(g + RB < ng)
        def _():
          start_gather(g + RB, r)

      return carry

    with jax.named_scope("mainloop"):
      lax.fori_loop(0, ng // RB, outer, 0)
    with jax.named_scope("drain"):
      for r in range(RB):
        wait_store(r)

  return sc_kernel(atoms2, edges_t)


def _tc_degree_linear(sc_sum, atoms2, deg_i8, w, b8, n_out, tile):
  d = sc_sum.shape[-1]
  ndeg = w.shape[0]

  def body(x_ref, a_ref, dg_ref, w_ref, b_ref, o_ref):
    dg = dg_ref[...]
    coeff = dg.astype(jnp.float32) - 5.0
    x = (x_ref[...] + coeff * a_ref[...]).astype(jnp.bfloat16)
    acc = jnp.zeros((tile, d), jnp.float32)
    for k in range(ndeg):
      y = lax.dot_general(x, w_ref[k].astype(jnp.bfloat16),
                          (((1,), (0,)), ((), ())),
                          preferred_element_type=jnp.float32)
      y = jnp.maximum(y + b_ref[k, :][None, :], 0.0)
      acc = acc + jnp.where(dg == k, y, 0.0)
    o_ref[...] = acc

  grid = (n_out + tile - 1) // tile
  return pl.pallas_call(
      body,
      grid=(grid,),
      in_specs=[
          pl.BlockSpec((tile, d), lambda i: (i, 0)),
          pl.BlockSpec((tile, d), lambda i: (i, 0)),
          pl.BlockSpec((tile, 1), lambda i: (i, 0)),
          pl.BlockSpec((ndeg, d, d), lambda i: (0, 0, 0)),
          pl.BlockSpec((8, d), lambda i: (0, 0)),
      ],
      out_specs=pl.BlockSpec((tile, d), lambda i: (i, 0)),
      out_shape=jax.ShapeDtypeStruct((n_out, d), jnp.float32),
  )(sc_sum, atoms2, deg_i8, w, b8)


def kernel(atoms, edges, W, b):
  bsz, n, d = atoms.shape
  k = edges.shape[-1]

  align = NW * GN * 8
  n_pad = ((n + align - 1) // align) * align
  tile = 1024
  assert bsz == 1

  atoms2 = atoms[0]
  e = edges[0]
  pad_cols = jnp.broadcast_to(
      (jnp.arange(n, n_pad, dtype=jnp.int32) % n)[None, :], (k, n_pad - n))
  edges_t = jnp.concatenate([e.T, pad_cols], axis=1)

  sc_sum = _sc_gather_sum(atoms2, edges_t, n_pad)

  deg = (e != -1).sum(-1).astype(jnp.int8)
  deg_i8 = jnp.pad(deg, (0, n_pad - n))[:, None]
  b8 = jnp.pad(b, ((0, 8 - b.shape[0]), (0, 0)))
  out = _tc_degree_linear(sc_sum, atoms2, deg_i8, W, b8, n, tile)
  return out[None]


# speedup vs baseline: 10.8854x; 1.1883x over previous
import functools

import jax
import jax.numpy as jnp
from jax import lax
from jax.experimental import pallas as pl
from jax.experimental.pallas import tpu as pltpu
from jax.experimental.pallas import tpu_sc as plsc

NC = 2
NS = 16
NW = NC * NS
GN = 16
RB = 4


def _sc_gather_sum(atoms2, edges_t, n_pad):
  d = atoms2.shape[-1]
  k = edges_t.shape[0]
  npw = n_pad // NW
  ng = npw // GN
  assert ng % RB == 0 and npw % 128 == 0
  gl = GN * k

  mesh = plsc.VectorSubcoreMesh(
      core_axis_name="c", subcore_axis_name="s", num_cores=NC, num_subcores=NS)

  @functools.partial(
      pl.kernel,
      out_type=jax.ShapeDtypeStruct((n_pad, d), jnp.float32),
      mesh=mesh,
      scratch_types=[
          pltpu.VMEM((k, npw), jnp.int32),
          pltpu.VMEM((ng * gl,), jnp.int32),
          pltpu.VMEM((RB, gl, d), jnp.float32),
          pltpu.VMEM((RB, GN, d), jnp.float32),
          pltpu.SemaphoreType.DMA((RB,)),
          pltpu.SemaphoreType.DMA((RB,)),
      ],
  )
  def sc_kernel(atoms_hbm, edges_hbm, out_hbm,
                edges_v, idx_v, rows_v, out_v, gsem, osem):
    c = lax.axis_index("c")
    s = lax.axis_index("s")
    wid = s * NC + c
    nbase = wid * npw

    with jax.named_scope("stage_edges"):
      pltpu.sync_copy(edges_hbm.at[:, pl.ds(nbase, npw)], edges_v)

    with jax.named_scope("assemble_idx"):
      def asm(g, carry):
        self16 = (jnp.full((16,), nbase + g * GN, jnp.int32)
                  + lax.iota(jnp.int32, 16))
        for kk in range(k):
          ev = edges_v[kk, pl.ds(g * GN, GN)]
          idx_v[pl.ds(g * gl + kk * GN, GN)] = jnp.where(ev < 0, self16, ev)
        return carry
      lax.fori_loop(0, ng, asm, 0, unroll=2)

    def start_gather(g, slot):
      pltpu.async_copy(atoms_hbm.at[idx_v.at[pl.ds(g * gl, gl)]],
                       rows_v.at[slot], gsem.at[slot])

    def start_store(g, slot):
      pltpu.async_copy(out_v.at[slot],
                       out_hbm.at[pl.ds(nbase + g * GN, GN)], osem.at[slot])

    def wait_gather(slot):
      pltpu.make_async_copy(atoms_hbm.at[idx_v.at[pl.ds(0, gl)]],
                            rows_v.at[slot], gsem.at[slot]).wait()

    def wait_store(slot):
      pltpu.make_async_copy(out_v.at[slot], out_hbm.at[pl.ds(nbase, GN)],
                            osem.at[slot]).wait()

    with jax.named_scope("prime_ring"):
      for r in range(RB):
        start_gather(r, r)

    def compute_group(g, slot):
      def node_body(i, carry):
        row = lambda kk, cc: rows_v[slot, kk * GN + i, pl.ds(cc * 16, 16)]
        for cc in range(d // 16):
          acc = row(0, cc) + row(1, cc)
          acc2 = row(2, cc) + row(3, cc)
          acc3 = row(4, cc) + row(5, cc)
          out_v[slot, i, pl.ds(cc * 16, 16)] = acc + (acc2 + acc3)
        return carry
      lax.fori_loop(0, GN, node_body, 0, unroll=2)

    def outer(o, carry):
      gg = o * RB
      for r in range(RB):
        g = gg + r
        wait_gather(r)

        @pl.when(o > 0)
        def _():
          wait_store(r)

        compute_group(g, r)
        start_store(g, r)

        @pl.when(g + RB < ng)
        def _():
          start_gather(g + RB, r)

      return carry

    with jax.named_scope("mainloop"):
      lax.fori_loop(0, ng // RB, outer, 0)
    with jax.named_scope("drain"):
      for r in range(RB):
        wait_store(r)

  return sc_kernel(atoms2, edges_t)


def _tc_degree_linear(sc_sum, atoms2, deg_i8, wfull, n_out, tile):
  d = sc_sum.shape[-1]

  def body(x_ref, a_ref, dg_ref, w_ref, o_ref):
    dg = dg_ref[...]
    coeff = dg.astype(jnp.float32) - 5.0
    x = (x_ref[...] + coeff * a_ref[...]).astype(jnp.bfloat16)
    dgb = jnp.broadcast_to(dg, (tile, d))
    parts = [jnp.where(dgb == float(k), x, jnp.bfloat16(0.0))
             for k in range(6)]
    lane = lax.broadcasted_iota(jnp.int32, (tile, d), 1).astype(jnp.bfloat16)
    parts.append((dgb == lane).astype(jnp.bfloat16))
    xcat = jnp.concatenate(parts, axis=1)
    acc = lax.dot_general(xcat, w_ref[...], (((1,), (0,)), ((), ())),
                          preferred_element_type=jnp.float32)
    o_ref[...] = jnp.maximum(acc, 0.0)

  grid = (n_out + tile - 1) // tile
  return pl.pallas_call(
      body,
      grid=(grid,),
      in_specs=[
          pl.BlockSpec((tile, d), lambda i: (i, 0)),
          pl.BlockSpec((tile, d), lambda i: (i, 0)),
          pl.BlockSpec((tile, 1), lambda i: (i, 0)),
          pl.BlockSpec((7 * d, d), lambda i: (0, 0)),
      ],
      out_specs=pl.BlockSpec((tile, d), lambda i: (i, 0)),
      out_shape=jax.ShapeDtypeStruct((n_out, d), jnp.float32),
  )(sc_sum, atoms2, deg_i8, wfull)


def kernel(atoms, edges, W, b):
  bsz, n, d = atoms.shape
  k = edges.shape[-1]

  align = NW * GN * 8
  n_pad = ((n + align - 1) // align) * align
  tile = 1024
  assert bsz == 1

  atoms2 = atoms[0]
  e = edges[0]
  pad_cols = jnp.broadcast_to(
      (jnp.arange(n, n_pad, dtype=jnp.int32) % n)[None, :], (k, n_pad - n))
  edges_t = jnp.concatenate([e.T, pad_cols], axis=1)

  sc_sum = _sc_gather_sum(atoms2, edges_t, n_pad)

  deg = (e != -1).sum(-1).astype(jnp.bfloat16)
  deg_i8 = jnp.pad(deg, (0, n_pad - n))[:, None]
  wfull = jnp.concatenate(
      [W.reshape(k * d, d), jnp.pad(b, ((0, d - b.shape[0]), (0, 0)))],
      axis=0).astype(jnp.bfloat16)
  out = _tc_degree_linear(sc_sum, atoms2, deg_i8, wfull, n, tile)
  return out[None]


# speedup vs baseline: 10.9051x; 1.0018x over previous
import functools

import jax
import jax.numpy as jnp
from jax import lax
from jax.experimental import pallas as pl
from jax.experimental.pallas import tpu as pltpu
from jax.experimental.pallas import tpu_sc as plsc

NC = 2
NS = 16
NW = NC * NS
GN = 16
RB = 4


def _sc_gather_sum(atoms2, edges_t, chunk, phase):
  d = atoms2.shape[-1]
  k = edges_t.shape[0]
  npw = chunk // NW
  ng = npw // GN
  assert ng % RB == 0 and npw % 128 == 0
  gl = GN * k

  mesh = plsc.VectorSubcoreMesh(
      core_axis_name="c", subcore_axis_name="s", num_cores=NC, num_subcores=NS)

  @functools.partial(
      pl.kernel,
      out_type=jax.ShapeDtypeStruct((chunk, d), jnp.float32),
      mesh=mesh,
      scratch_types=[
          pltpu.VMEM((k, npw), jnp.int32),
          pltpu.VMEM((ng * gl,), jnp.int32),
          pltpu.VMEM((RB, gl, d), jnp.float32),
          pltpu.VMEM((RB, GN, d), jnp.float32),
          pltpu.SemaphoreType.DMA((RB,)),
          pltpu.SemaphoreType.DMA((RB,)),
      ],
  )
  def sc_kernel(atoms_hbm, edges_hbm, out_hbm,
                edges_v, idx_v, rows_v, out_v, gsem, osem):
    c = lax.axis_index("c")
    s = lax.axis_index("s")
    wid = s * NC + c
    nbase = wid * npw
    gbase = phase * chunk + nbase

    with jax.named_scope("stage_edges"):
      pltpu.sync_copy(edges_hbm.at[:, pl.ds(gbase, npw)], edges_v)

    with jax.named_scope("assemble_idx"):
      def asm(g, carry):
        self16 = (jnp.full((16,), gbase + g * GN, jnp.int32)
                  + lax.iota(jnp.int32, 16))
        for kk in range(k):
          ev = edges_v[kk, pl.ds(g * GN, GN)]
          idx_v[pl.ds(g * gl + kk * GN, GN)] = jnp.where(ev < 0, self16, ev)
        return carry
      lax.fori_loop(0, ng, asm, 0, unroll=2)

    def start_gather(g, slot):
      pltpu.async_copy(atoms_hbm.at[idx_v.at[pl.ds(g * gl, gl)]],
                       rows_v.at[slot], gsem.at[slot])

    def start_store(g, slot):
      pltpu.async_copy(out_v.at[slot],
                       out_hbm.at[pl.ds(nbase + g * GN, GN)], osem.at[slot])

    def wait_gather(slot):
      pltpu.make_async_copy(atoms_hbm.at[idx_v.at[pl.ds(0, gl)]],
                            rows_v.at[slot], gsem.at[slot]).wait()

    def wait_store(slot):
      pltpu.make_async_copy(out_v.at[slot], out_hbm.at[pl.ds(nbase, GN)],
                            osem.at[slot]).wait()

    with jax.named_scope("prime_ring"):
      for r in range(RB):
        start_gather(r, r)

    def compute_group(g, slot):
      def node_body(i, carry):
        row = lambda kk, cc: rows_v[slot, kk * GN + i, pl.ds(cc * 16, 16)]
        for cc in range(d // 16):
          acc = row(0, cc) + row(1, cc)
          acc2 = row(2, cc) + row(3, cc)
          acc3 = row(4, cc) + row(5, cc)
          out_v[slot, i, pl.ds(cc * 16, 16)] = acc + (acc2 + acc3)
        return carry
      lax.fori_loop(0, GN, node_body, 0, unroll=2)

    def outer(o, carry):
      gg = o * RB
      for r in range(RB):
        g = gg + r
        wait_gather(r)

        @pl.when(o > 0)
        def _():
          wait_store(r)

        compute_group(g, r)
        start_store(g, r)

        @pl.when(g + RB < ng)
        def _():
          start_gather(g + RB, r)

      return carry

    with jax.named_scope("mainloop"):
      lax.fori_loop(0, ng // RB, outer, 0)
    with jax.named_scope("drain"):
      for r in range(RB):
        wait_store(r)

  return sc_kernel(atoms2, edges_t)


def _tc_degree_linear(outbuf, sc_sum, atoms2, deg_i8, wfull, n_out, tile,
                      chunk, phase):
  d = sc_sum.shape[-1]

  def body(x_ref, a_ref, dg_ref, w_ref, o_ref):
    dg = dg_ref[...]
    coeff = dg.astype(jnp.float32) - 5.0
    x = (x_ref[...] + coeff * a_ref[...]).astype(jnp.bfloat16)
    dgb = jnp.broadcast_to(dg, (tile, d))
    parts = [jnp.where(dgb == float(k), x, jnp.bfloat16(0.0))
             for k in range(6)]
    lane = lax.broadcasted_iota(jnp.int32, (tile, d), 1).astype(jnp.bfloat16)
    parts.append((dgb == lane).astype(jnp.bfloat16))
    xcat = jnp.concatenate(parts, axis=1)
    acc = lax.dot_general(xcat, w_ref[...], (((1,), (0,)), ((), ())),
                          preferred_element_type=jnp.float32)
    o_ref[...] = jnp.maximum(acc, 0.0)

  poff = phase * (chunk // tile)
  nrows = min(chunk, n_out - phase * chunk)
  grid = (nrows + tile - 1) // tile
  in_specs = [
      pl.BlockSpec((tile, d), lambda i: (i, 0)),
      pl.BlockSpec((tile, d), lambda i: (poff + i, 0)),
      pl.BlockSpec((tile, 1), lambda i: (poff + i, 0)),
      pl.BlockSpec((7 * d, d), lambda i: (0, 0)),
  ]
  args = [sc_sum, atoms2, deg_i8, wfull]
  aliases = {}
  fn = body
  if outbuf is not None:
    in_specs.insert(0, pl.BlockSpec(memory_space=pltpu.ANY))
    args.insert(0, outbuf)
    aliases = {0: 0}
    fn = lambda _, *refs: body(*refs)
  return pl.pallas_call(
      fn,
      grid=(grid,),
      in_specs=in_specs,
      out_specs=pl.BlockSpec((tile, d), lambda i: (poff + i, 0)),
      out_shape=jax.ShapeDtypeStruct((n_out, d), jnp.float32),
      input_output_aliases=aliases,
  )(*args)


def kernel(atoms, edges, W, b):
  bsz, n, d = atoms.shape
  k = edges.shape[-1]

  align = NW * GN * 8
  n_pad = ((n + align - 1) // align) * align
  tile = 1024
  assert bsz == 1

  atoms2 = atoms[0]
  e = edges[0]
  pad_cols = jnp.broadcast_to(
      (jnp.arange(n, n_pad, dtype=jnp.int32) % n)[None, :], (k, n_pad - n))
  edges_t = jnp.concatenate([e.T, pad_cols], axis=1)

  deg = (e != -1).sum(-1).astype(jnp.bfloat16)
  deg_i8 = jnp.pad(deg, (0, n_pad - n))[:, None]
  wfull = jnp.concatenate(
      [W.reshape(k * d, d), jnp.pad(b, ((0, d - b.shape[0]), (0, 0)))],
      axis=0).astype(jnp.bfloat16)

  phases = 1
  chunk = n_pad // phases
  assert chunk % tile == 0
  out = None
  for p in range(phases):
    sc_p = _sc_gather_sum(atoms2, edges_t, chunk, p)
    out = _tc_degree_linear(out, sc_p, atoms2, deg_i8, wfull, n, tile,
                            chunk, p)
  return out[None]
